# baseline (speedup 1.0000x reference)
import functools

import jax
import jax.numpy as jnp
from jax import lax
from jax.experimental import pallas as pl
from jax.experimental.pallas import tpu as pltpu
from jax.experimental.pallas import tpu_sc as plsc

HIDDEN = 1024
VOCAB = 32768
TOPK = 64
SEQ = 2048
NW = 32
TPW = SEQ // NW


def _mm_kernel(x_ref, w_ref, b_ref, o_ref):
    acc = jax.lax.dot_general(
        x_ref[...], w_ref[...], (((1,), (1,)), ((), ())),
        preferred_element_type=jnp.float32)
    o_ref[...] = jnp.maximum(acc + b_ref[...], 0.0)


def _activations(x2d, W_to, b_to):
    S = x2d.shape[0]
    SB, VB = 256, 1024
    return pl.pallas_call(
        _mm_kernel,
        grid=(S // SB, VOCAB // VB),
        in_specs=[
            pl.BlockSpec((SB, HIDDEN), lambda i, j: (i, 0)),
            pl.BlockSpec((VB, HIDDEN), lambda i, j: (j, 0)),
            pl.BlockSpec((1, VB), lambda i, j: (0, j)),
        ],
        out_specs=pl.BlockSpec((SB, VB), lambda i, j: (i, j)),
        out_shape=jax.ShapeDtypeStruct((S, VOCAB), jnp.float32),
    )(x2d, W_to, b_to.reshape(1, VOCAB))


def _s16(k, i):
    return plsc.sort_key_val(k, i, descending=True)


def _rv(v):
    return lax.rev(v, (0,))


def _ce(a, b):
    sel = a[0] >= b[0]
    mx = (jnp.where(sel, a[0], b[0]), jnp.where(sel, a[1], b[1]))
    mn = (jnp.where(sel, b[0], a[0]), jnp.where(sel, b[1], a[1]))
    return mx, mn


def _merge_16_16(a, b):
    rb = (_rv(b[0]), _rv(b[1]))
    hi, lo = _ce(a, rb)
    return [_s16(*hi), _s16(*lo)]


def _merge_32_32(A, B):
    rb = [(_rv(B[1][0]), _rv(B[1][1])), (_rv(B[0][0]), _rv(B[0][1]))]
    h0, l0 = _ce(A[0], rb[0])
    h1, l1 = _ce(A[1], rb[1])
    hh, hl = _ce(h0, h1)
    lh, ll = _ce(l0, l1)
    return [_s16(*hh), _s16(*hl), _s16(*lh), _s16(*ll)]


def _build64(kv, iv):
    s = [_s16(kv[i], iv[i]) for i in range(4)]
    m0 = _merge_16_16(s[0], s[1])
    m1 = _merge_16_16(s[2], s[3])
    return _merge_32_32(m0, m1)


def _merge_64_64_top(A, B):
    rb = [(_rv(B[3 - i][0]), _rv(B[3 - i][1])) for i in range(4)]
    h = [_ce(A[i], rb[i])[0] for i in range(4)]
    h02, h2_ = _ce(h[0], h[2])
    h13, h3_ = _ce(h[1], h[3])
    a, b_ = _ce(h02, h13)
    c, d = _ce(h2_, h3_)
    return [_s16(*a), _s16(*b_), _s16(*c), _s16(*d)]


def _list_load(lk, li, slot):
    return [(lk[pl.ds((slot * 4 + i) * 16, 16)],
             li[pl.ds((slot * 4 + i) * 16, 16)]) for i in range(4)]


def _list_store(lk, li, slot, L):
    for i in range(4):
        lk[pl.ds((slot * 4 + i) * 16, 16)] = L[i][0]
        li[pl.ds((slot * 4 + i) * 16, 16)] = L[i][1]


def _topk_body(tpw, act, vals_out, idx_out, row, cm, lk, li, ek, ei, ov, oi):
    wid = lax.axis_index("s") * 2 + lax.axis_index("c")
    iota = lax.iota(jnp.int32, 16)
    TPW = tpw

    def token_body(tl, carry):
        token = wid * TPW + tl
        pltpu.sync_copy(act.at[token], row)

        def l1(g, c):
            m = row[pl.ds(g * 256, 16)]
            for r in range(1, 16):
                m = jnp.maximum(m, row[pl.ds(g * 256 + r * 16, 16)])
            cm[pl.ds(g * 16, 16)] = m
            return c
        lax.fori_loop(0, 128, l1, 0, unroll=2)

        def p1(q, c):
            kv = [cm[pl.ds((q * 4 + i) * 16, 16)] for i in range(4)]
            iv = [(q * 4 + i) * 16 + iota for i in range(4)]
            _list_store(lk, li, q, _build64(kv, iv))
            return c
        lax.fori_loop(0, 32, p1, 0, unroll=2)
        for nm in (16, 8, 4, 2, 1):
            def p2(j, c, nm=nm):
                M = _merge_64_64_top(_list_load(lk, li, 2 * j),
                                     _list_load(lk, li, 2 * j + 1))
                _list_store(lk, li, j, M)
                return c
            lax.fori_loop(0, nm, p2, 0)

        def ep(v, c):
            cid = li[pl.ds(v * 16, 16)]
            base = lax.shift_right_logical(cid, 4) * 256 + (cid & 15)
            for r in range(16):
                pos = base + 16 * r
                ek[pl.ds((v * 16 + r) * 16, 16)] = plsc.load_gather(row, [pos])
                ei[pl.ds((v * 16 + r) * 16, 16)] = pos
            return c
        lax.fori_loop(0, 4, ep, 0)

        def p1b(q, c):
            kv = [ek[pl.ds((q * 4 + i) * 16, 16)] for i in range(4)]
            iv = [ei[pl.ds((q * 4 + i) * 16, 16)] for i in range(4)]
            _list_store(lk, li, q, _build64(kv, iv))
            return c
        lax.fori_loop(0, 16, p1b, 0, unroll=2)
        for nm in (8, 4, 2, 1):
            def p2b(j, c, nm=nm):
                M = _merge_64_64_top(_list_load(lk, li, 2 * j),
                                     _list_load(lk, li, 2 * j + 1))
                _list_store(lk, li, j, M)
                return c
            lax.fori_loop(0, nm, p2b, 0)

        for i in range(4):
            ov[pl.ds(tl * 64 + i * 16, 16)] = lk[pl.ds(i * 16, 16)]
            oi[pl.ds(tl * 64 + i * 16, 16)] = li[pl.ds(i * 16, 16)]
        return carry

    lax.fori_loop(0, TPW, token_body, 0)
    pltpu.sync_copy(ov, vals_out.at[pl.ds(wid * TPW * 64, TPW * 64)])
    pltpu.sync_copy(oi, idx_out.at[pl.ds(wid * TPW * 64, TPW * 64)])


def _topk_sc(act):
    S = act.shape[0]
    tpw = S // NW
    mesh = plsc.VectorSubcoreMesh(core_axis_name="c", subcore_axis_name="s")
    fn = functools.partial(
        pl.kernel,
        mesh=mesh,
        compiler_params=pltpu.CompilerParams(needs_layout_passes=False),
        out_type=[
            jax.ShapeDtypeStruct((S * TOPK,), jnp.float32),
            jax.ShapeDtypeStruct((S * TOPK,), jnp.int32),
        ],
        scratch_types=[
            pltpu.VMEM((VOCAB,), jnp.float32),
            pltpu.VMEM((2048,), jnp.float32),
            pltpu.VMEM((2048,), jnp.float32),
            pltpu.VMEM((2048,), jnp.int32),
            pltpu.VMEM((1024,), jnp.float32),
            pltpu.VMEM((1024,), jnp.int32),
            pltpu.VMEM((tpw * 64,), jnp.float32),
            pltpu.VMEM((tpw * 64,), jnp.int32),
        ],
    )(functools.partial(_topk_body, tpw))
    return fn(act)


IPW = SEQ * TOPK // NW
GB = 32


def _gather_body(ipw, wto, wft, idx, protos_out, wsel_out, idbuf, buf1, buf2,
                 sem1, sem2, sem3, sem4):
    wid = lax.axis_index("s") * 2 + lax.axis_index("c")
    IPW = ipw

    def batch(b, c):
        base = wid * IPW + b * GB
        pltpu.sync_copy(idx.at[pl.ds(base, GB)], idbuf)
        c1 = pltpu.async_copy(wto.at[idbuf], buf1, sem1)
        c2 = pltpu.async_copy(wft.at[idbuf], buf2, sem2)
        c1.wait()
        c3 = pltpu.async_copy(buf1, protos_out.at[pl.ds(base, GB)], sem3)
        c2.wait()
        c4 = pltpu.async_copy(buf2, wsel_out.at[pl.ds(base, GB)], sem4)
        c3.wait()
        c4.wait()
        return c
    lax.fori_loop(0, IPW // GB, batch, 0)


def _gather_sc(W_to, W_from_t, idx_f):
    N = idx_f.shape[0]
    ipw = N // NW
    mesh = plsc.VectorSubcoreMesh(core_axis_name="c", subcore_axis_name="s")
    fn = functools.partial(
        pl.kernel,
        mesh=mesh,
        compiler_params=pltpu.CompilerParams(needs_layout_passes=False),
        out_type=[
            jax.ShapeDtypeStruct((N, HIDDEN), jnp.float32),
            jax.ShapeDtypeStruct((N, HIDDEN), jnp.float32),
        ],
        scratch_types=[
            pltpu.VMEM((GB,), jnp.int32),
            pltpu.VMEM((GB, HIDDEN), jnp.float32),
            pltpu.VMEM((GB, HIDDEN), jnp.float32),
            pltpu.SemaphoreType.DMA,
            pltpu.SemaphoreType.DMA,
            pltpu.SemaphoreType.DMA,
            pltpu.SemaphoreType.DMA,
        ],
    )(functools.partial(_gather_body, ipw))
    return fn(W_to, W_from_t, idx_f)


TB = 16


def _gram_kernel(x_ref, vals_ref, protos_ref, wsel_ref, bfrom_ref, alpha_ref,
                 o_ref):
    a = alpha_ref[0]
    ii = lax.broadcasted_iota(jnp.int32, (TOPK, TOPK), 0)
    jj = lax.broadcasted_iota(jnp.int32, (TOPK, TOPK), 1)
    for t in range(TB):
        P = protos_ref[pl.ds(t * TOPK, TOPK), :]
        n = jnp.sqrt(jnp.sum(P * P, axis=1, keepdims=True))
        Pn = P / jnp.maximum(n, 1e-12)
        G = jax.lax.dot_general(Pn, Pn, (((1,), (1,)), ((), ())),
                                preferred_element_type=jnp.float32)
        G = jnp.where(ii == jj, G - 1.0, G)
        G = jnp.maximum(G, 0.0)
        v = vals_ref[pl.ds(t, 1), :]
        w = jax.nn.softmax(v, axis=-1)
        inh = jax.lax.dot_general(w, G, (((1,), (0,)), ((), ())),
                                  preferred_element_type=jnp.float32)
        r = jnp.maximum(v * (1.0 - a * inh), 0.0)
        W = wsel_ref[pl.ds(t * TOPK, TOPK), :]
        out_t = jax.lax.dot_general(r, W, (((1,), (0,)), ((), ())),
                                    preferred_element_type=jnp.float32)
        o_ref[pl.ds(t, 1), :] = x_ref[pl.ds(t, 1), :] + out_t + bfrom_ref[...]


def _gram_combine(x2d, vals, protos, wsel, b_from, alpha):
    S = x2d.shape[0]
    return pl.pallas_call(
        _gram_kernel,
        grid=(S // TB,),
        in_specs=[
            pl.BlockSpec((TB, HIDDEN), lambda i: (i, 0)),
            pl.BlockSpec((TB, TOPK), lambda i: (i, 0)),
            pl.BlockSpec((TB * TOPK, HIDDEN), lambda i: (i, 0)),
            pl.BlockSpec((TB * TOPK, HIDDEN), lambda i: (i, 0)),
            pl.BlockSpec((1, HIDDEN), lambda i: (0, 0)),
            pl.BlockSpec(memory_space=pltpu.SMEM),
        ],
        out_specs=pl.BlockSpec((TB, HIDDEN), lambda i: (i, 0)),
        out_shape=jax.ShapeDtypeStruct((S, HIDDEN), jnp.float32),
    )(x2d, vals, protos, wsel, b_from.reshape(1, HIDDEN),
      alpha.reshape(1))


NSLICE = 8


def kernel(x, W_to, b_to, W_from, b_from, alpha):
    B, S, H = x.shape
    x2d = x.reshape(B * S, H)
    W_from_t = W_from.T
    SL = S // NSLICE
    outs = []
    for n in range(NSLICE):
        xs = x2d[n * SL:(n + 1) * SL]
        act = _activations(xs, W_to, b_to)
        vals_f, idx_f = _topk_sc(act)
        protos, wsel = _gather_sc(W_to, W_from_t, idx_f)
        outs.append(_gram_combine(xs, vals_f.reshape(SL, TOPK), protos,
                                  wsel, b_from, alpha))
    out = jnp.concatenate(outs, axis=0)
    return out.reshape(B, S, H)


# speedup vs baseline: 1.0386x; 1.0386x over previous
import functools

import jax
import jax.numpy as jnp
from jax import lax
from jax.experimental import pallas as pl
from jax.experimental.pallas import tpu as pltpu
from jax.experimental.pallas import tpu_sc as plsc

HIDDEN = 1024
VOCAB = 32768
TOPK = 64
SEQ = 2048
NW = 32
TPW = SEQ // NW


def _mm_kernel(x_ref, w_ref, b_ref, o_ref):
    acc = jax.lax.dot_general(
        x_ref[...], w_ref[...], (((1,), (1,)), ((), ())),
        preferred_element_type=jnp.float32)
    o_ref[...] = jnp.maximum(acc + b_ref[...], 0.0)


def _activations(x2d, W_to, b_to):
    S = x2d.shape[0]
    SB, VB = 256, 1024
    return pl.pallas_call(
        _mm_kernel,
        grid=(S // SB, VOCAB // VB),
        in_specs=[
            pl.BlockSpec((SB, HIDDEN), lambda i, j: (i, 0)),
            pl.BlockSpec((VB, HIDDEN), lambda i, j: (j, 0)),
            pl.BlockSpec((1, VB), lambda i, j: (0, j)),
        ],
        out_specs=pl.BlockSpec((SB, VB), lambda i, j: (i, j)),
        out_shape=jax.ShapeDtypeStruct((S, VOCAB), jnp.float32),
    )(x2d, W_to, b_to.reshape(1, VOCAB))


def _s16(k, i):
    return plsc.sort_key_val(k, i, descending=True)


def _rv(v):
    return lax.rev(v, (0,))


def _ce(a, b):
    sel = a[0] >= b[0]
    mx = (jnp.where(sel, a[0], b[0]), jnp.where(sel, a[1], b[1]))
    mn = (jnp.where(sel, b[0], a[0]), jnp.where(sel, b[1], a[1]))
    return mx, mn


def _merge_16_16(a, b):
    rb = (_rv(b[0]), _rv(b[1]))
    hi, lo = _ce(a, rb)
    return [_s16(*hi), _s16(*lo)]


def _merge_32_32(A, B):
    rb = [(_rv(B[1][0]), _rv(B[1][1])), (_rv(B[0][0]), _rv(B[0][1]))]
    h0, l0 = _ce(A[0], rb[0])
    h1, l1 = _ce(A[1], rb[1])
    hh, hl = _ce(h0, h1)
    lh, ll = _ce(l0, l1)
    return [_s16(*hh), _s16(*hl), _s16(*lh), _s16(*ll)]


def _build64(kv, iv):
    s = [_s16(kv[i], iv[i]) for i in range(4)]
    m0 = _merge_16_16(s[0], s[1])
    m1 = _merge_16_16(s[2], s[3])
    return _merge_32_32(m0, m1)


def _merge_64_64_top(A, B):
    rb = [(_rv(B[3 - i][0]), _rv(B[3 - i][1])) for i in range(4)]
    h = [_ce(A[i], rb[i])[0] for i in range(4)]
    h02, h2_ = _ce(h[0], h[2])
    h13, h3_ = _ce(h[1], h[3])
    a, b_ = _ce(h02, h13)
    c, d = _ce(h2_, h3_)
    return [_s16(*a), _s16(*b_), _s16(*c), _s16(*d)]


def _list_load(lk, li, slot):
    return [(lk[pl.ds((slot * 4 + i) * 16, 16)],
             li[pl.ds((slot * 4 + i) * 16, 16)]) for i in range(4)]


def _list_store(lk, li, slot, L):
    for i in range(4):
        lk[pl.ds((slot * 4 + i) * 16, 16)] = L[i][0]
        li[pl.ds((slot * 4 + i) * 16, 16)] = L[i][1]


def _topk_body(tpw, act, vals_out, idx_out, row, cm, lk, li, ek, ei, ov, oi):
    wid = lax.axis_index("s") * 2 + lax.axis_index("c")
    iota = lax.iota(jnp.int32, 16)
    TPW = tpw

    def token_body(tl, carry):
        token = wid * TPW + tl
        pltpu.sync_copy(act.at[token], row)

        def l1(g, c):
            m = row[pl.ds(g * 256, 16)]
            for r in range(1, 16):
                m = jnp.maximum(m, row[pl.ds(g * 256 + r * 16, 16)])
            cm[pl.ds(g * 16, 16)] = m
            return c
        lax.fori_loop(0, 128, l1, 0, unroll=2)

        def p1(q, c):
            kv = [cm[pl.ds((q * 4 + i) * 16, 16)] for i in range(4)]
            iv = [(q * 4 + i) * 16 + iota for i in range(4)]
            _list_store(lk, li, q, _build64(kv, iv))
            return c
        lax.fori_loop(0, 32, p1, 0, unroll=2)
        for nm in (16, 8, 4, 2, 1):
            def p2(j, c, nm=nm):
                M = _merge_64_64_top(_list_load(lk, li, 2 * j),
                                     _list_load(lk, li, 2 * j + 1))
                _list_store(lk, li, j, M)
                return c
            lax.fori_loop(0, nm, p2, 0)

        def ep(v, c):
            cid = li[pl.ds(v * 16, 16)]
            base = lax.shift_right_logical(cid, 4) * 256 + (cid & 15)
            for r in range(16):
                pos = base + 16 * r
                ek[pl.ds((v * 16 + r) * 16, 16)] = plsc.load_gather(row, [pos])
                ei[pl.ds((v * 16 + r) * 16, 16)] = pos
            return c
        lax.fori_loop(0, 4, ep, 0)

        def p1b(q, c):
            kv = [ek[pl.ds((q * 4 + i) * 16, 16)] for i in range(4)]
            iv = [ei[pl.ds((q * 4 + i) * 16, 16)] for i in range(4)]
            _list_store(lk, li, q, _build64(kv, iv))
            return c
        lax.fori_loop(0, 16, p1b, 0, unroll=2)
        for nm in (8, 4, 2, 1):
            def p2b(j, c, nm=nm):
                M = _merge_64_64_top(_list_load(lk, li, 2 * j),
                                     _list_load(lk, li, 2 * j + 1))
                _list_store(lk, li, j, M)
                return c
            lax.fori_loop(0, nm, p2b, 0)

        for i in range(4):
            ov[pl.ds(tl * 64 + i * 16, 16)] = lk[pl.ds(i * 16, 16)]
            oi[pl.ds(tl * 64 + i * 16, 16)] = li[pl.ds(i * 16, 16)]
        return carry

    lax.fori_loop(0, TPW, token_body, 0)
    pltpu.sync_copy(ov, vals_out.at[pl.ds(wid * TPW * 64, TPW * 64)])
    pltpu.sync_copy(oi, idx_out.at[pl.ds(wid * TPW * 64, TPW * 64)])


def _topk_sc(act):
    S = act.shape[0]
    tpw = S // NW
    mesh = plsc.VectorSubcoreMesh(core_axis_name="c", subcore_axis_name="s")
    fn = functools.partial(
        pl.kernel,
        mesh=mesh,
        compiler_params=pltpu.CompilerParams(needs_layout_passes=False),
        out_type=[
            jax.ShapeDtypeStruct((S * TOPK,), jnp.float32),
            jax.ShapeDtypeStruct((S * TOPK,), jnp.int32),
        ],
        scratch_types=[
            pltpu.VMEM((VOCAB,), jnp.float32),
            pltpu.VMEM((2048,), jnp.float32),
            pltpu.VMEM((2048,), jnp.float32),
            pltpu.VMEM((2048,), jnp.int32),
            pltpu.VMEM((1024,), jnp.float32),
            pltpu.VMEM((1024,), jnp.int32),
            pltpu.VMEM((tpw * 64,), jnp.float32),
            pltpu.VMEM((tpw * 64,), jnp.int32),
        ],
    )(functools.partial(_topk_body, tpw))
    return fn(act)


IPW = SEQ * TOPK // NW
GB = 64


def _gather_body(ipw, wto, wft, idx, protos_out, wsel_out, idbuf, buf, sem):
    wid = lax.axis_index("s") * 2 + lax.axis_index("c")
    IPW = ipw

    def batch(b, c):
        base = wid * IPW + b * GB
        pltpu.sync_copy(idx.at[pl.ds(base, GB)], idbuf)
        pltpu.async_copy(wto.at[idbuf], buf, sem).wait()
        pltpu.sync_copy(buf, protos_out.at[pl.ds(base, GB)])
        pltpu.async_copy(wft.at[idbuf], buf, sem).wait()
        pltpu.sync_copy(buf, wsel_out.at[pl.ds(base, GB)])
        return c
    lax.fori_loop(0, IPW // GB, batch, 0)


def _gather_sc(W_to, W_from_t, idx_f):
    N = idx_f.shape[0]
    ipw = N // NW
    mesh = plsc.VectorSubcoreMesh(core_axis_name="c", subcore_axis_name="s")
    fn = functools.partial(
        pl.kernel,
        mesh=mesh,
        compiler_params=pltpu.CompilerParams(needs_layout_passes=False),
        out_type=[
            jax.ShapeDtypeStruct((N, HIDDEN), jnp.float32),
            jax.ShapeDtypeStruct((N, HIDDEN), jnp.float32),
        ],
        scratch_types=[
            pltpu.VMEM((GB,), jnp.int32),
            pltpu.VMEM((GB, HIDDEN), jnp.float32),
            pltpu.SemaphoreType.DMA,
        ],
    )(functools.partial(_gather_body, ipw))
    return fn(W_to, W_from_t, idx_f)


TB = 16


def _gram_kernel(x_ref, vals_ref, protos_ref, wsel_ref, bfrom_ref, alpha_ref,
                 o_ref):
    a = alpha_ref[0]
    ii = lax.broadcasted_iota(jnp.int32, (TOPK, TOPK), 0)
    jj = lax.broadcasted_iota(jnp.int32, (TOPK, TOPK), 1)
    for t in range(TB):
        P = protos_ref[pl.ds(t * TOPK, TOPK), :]
        n = jnp.sqrt(jnp.sum(P * P, axis=1, keepdims=True))
        Pn = P / jnp.maximum(n, 1e-12)
        G = jax.lax.dot_general(Pn, Pn, (((1,), (1,)), ((), ())),
                                preferred_element_type=jnp.float32)
        G = jnp.where(ii == jj, G - 1.0, G)
        G = jnp.maximum(G, 0.0)
        v = vals_ref[pl.ds(t, 1), :]
        w = jax.nn.softmax(v, axis=-1)
        inh = jax.lax.dot_general(w, G, (((1,), (0,)), ((), ())),
                                  preferred_element_type=jnp.float32)
        r = jnp.maximum(v * (1.0 - a * inh), 0.0)
        W = wsel_ref[pl.ds(t * TOPK, TOPK), :]
        out_t = jax.lax.dot_general(r, W, (((1,), (0,)), ((), ())),
                                    preferred_element_type=jnp.float32)
        o_ref[pl.ds(t, 1), :] = x_ref[pl.ds(t, 1), :] + out_t + bfrom_ref[...]


def _gram_combine(x2d, vals, protos, wsel, b_from, alpha):
    S = x2d.shape[0]
    return pl.pallas_call(
        _gram_kernel,
        grid=(S // TB,),
        in_specs=[
            pl.BlockSpec((TB, HIDDEN), lambda i: (i, 0)),
            pl.BlockSpec((TB, TOPK), lambda i: (i, 0)),
            pl.BlockSpec((TB * TOPK, HIDDEN), lambda i: (i, 0)),
            pl.BlockSpec((TB * TOPK, HIDDEN), lambda i: (i, 0)),
            pl.BlockSpec((1, HIDDEN), lambda i: (0, 0)),
            pl.BlockSpec(memory_space=pltpu.SMEM),
        ],
        out_specs=pl.BlockSpec((TB, HIDDEN), lambda i: (i, 0)),
        out_shape=jax.ShapeDtypeStruct((S, HIDDEN), jnp.float32),
    )(x2d, vals, protos, wsel, b_from.reshape(1, HIDDEN),
      alpha.reshape(1))


def _tr_kernel(w_ref, o_ref):
    o_ref[...] = w_ref[...].T


def _transpose_tc(W_from):
    HB, VB = 256, 2048
    return pl.pallas_call(
        _tr_kernel,
        grid=(VOCAB // VB, HIDDEN // HB),
        in_specs=[pl.BlockSpec((HB, VB), lambda i, j: (j, i))],
        out_specs=pl.BlockSpec((VB, HB), lambda i, j: (i, j)),
        out_shape=jax.ShapeDtypeStruct((VOCAB, HIDDEN), jnp.float32),
    )(W_from)


NSLICE = 8


def kernel(x, W_to, b_to, W_from, b_from, alpha):
    B, S, H = x.shape
    x2d = x.reshape(B * S, H)
    W_from_t = _transpose_tc(W_from)
    SL = S // NSLICE
    outs = []
    for n in range(NSLICE):
        xs = x2d[n * SL:(n + 1) * SL]
        act = _activations(xs, W_to, b_to)
        vals_f, idx_f = _topk_sc(act)
        protos, wsel = _gather_sc(W_to, W_from_t, idx_f)
        outs.append(_gram_combine(xs, vals_f.reshape(SL, TOPK), protos,
                                  wsel, b_from, alpha))
    out = jnp.concatenate(outs, axis=0)
    return out.reshape(B, S, H)
